# baseline (speedup 1.0000x reference)
import jax
import jax.numpy as jnp
from jax import lax
from jax.experimental import pallas as pl
from jax.experimental.pallas import tpu as pltpu
from jax.experimental.pallas import tpu_sc as plsc

B = 16384
L = 32
NC = 2
NS = 16
NW = NC * NS
RPW = B // NW
NCHUNK = 4
RC = RPW // NCHUNK
IC = RC * L


def _repack(p2d, idx1d):
    def r_body(r, carry):
        for cb in range(L // 16):
            idx1d[pl.ds(r * L + cb * 16, 16)] = p2d[r, pl.ds(cb * 16, 16)]
        return carry
    lax.fori_loop(0, RC, r_body, 0)


def _body(x_hbm, w_hbm, p_hbm, out_hbm,
          p2d0, p2d1, idx0, idx1, gath0, gath1, x_v, out_v, sem0, sem1):
    wid = lax.axis_index("s") * NC + lax.axis_index("c")
    base = wid * RPW

    p2ds = [p2d0, p2d1]
    idxs = [idx0, idx1]
    gaths = [gath0, gath1]
    sems = [sem0, sem1]
    copies = [None, None]

    pltpu.sync_copy(x_hbm.at[pl.ds(base, RPW)], x_v)
    pltpu.sync_copy(p_hbm.at[pl.ds(base, RC)], p2d0)
    _repack(p2d0, idx0)
    copies[0] = pltpu.async_copy(w_hbm.at[idx0], gath0, sem0)

    lane = lax.broadcasted_iota(jnp.int32, (16,), 0) * L

    for c in range(NCHUNK):
        cur = c % 2
        nxt = (c + 1) % 2
        if c + 1 < NCHUNK:
            pltpu.sync_copy(
                p_hbm.at[pl.ds(base + (c + 1) * RC, RC)], p2ds[nxt])
            _repack(p2ds[nxt], idxs[nxt])
            copies[nxt] = pltpu.async_copy(
                w_hbm.at[idxs[nxt]], gaths[nxt], sems[nxt])
        copies[cur].wait()
        gbuf = gaths[cur]

        def g_body(g, carry):
            off = g * (16 * L)
            row0 = pl.multiple_of(c * RC + g * 16, 16)
            acc = x_v[pl.ds(row0, 16)]
            for l in range(L):
                acc = acc * plsc.load_gather(gbuf, [lane + (off + l)])
            out_v[pl.ds(row0, 16)] = acc
            return carry

        lax.fori_loop(0, RC // 16, g_body, 0)

    pltpu.sync_copy(out_v, out_hbm.at[pl.ds(base, RPW)])


def kernel(x, weights, paths):
    paths32 = paths.astype(jnp.int32)
    mesh = plsc.VectorSubcoreMesh(core_axis_name="c", subcore_axis_name="s")
    f = pl.kernel(
        _body,
        out_type=jax.ShapeDtypeStruct((B,), jnp.float32),
        mesh=mesh,
        scratch_types=[
            pltpu.VMEM((RC, L), jnp.int32),
            pltpu.VMEM((RC, L), jnp.int32),
            pltpu.VMEM((IC,), jnp.int32),
            pltpu.VMEM((IC,), jnp.int32),
            pltpu.VMEM((IC,), jnp.float32),
            pltpu.VMEM((IC,), jnp.float32),
            pltpu.VMEM((RPW,), jnp.float32),
            pltpu.VMEM((RPW,), jnp.float32),
            pltpu.SemaphoreType.DMA,
            pltpu.SemaphoreType.DMA,
        ],
        compiler_params=pltpu.CompilerParams(
            needs_layout_passes=False, use_tc_tiling_on_sc=True),
    )
    return f(x, weights, paths32)


# speedup vs baseline: 1.1746x; 1.1746x over previous
import jax
import jax.numpy as jnp
from jax import lax
from jax.experimental import pallas as pl
from jax.experimental.pallas import tpu as pltpu
from jax.experimental.pallas import tpu_sc as plsc

B = 16384
L = 32
NC = 2
NS = 16
NW = NC * NS
RPW = B // NW
NCHUNK = 4
RC = RPW // NCHUNK
IC = RC * L


def _body(x_hbm, w_hbm, pt_hbm, out_hbm,
          idx0, idx1, gath0, gath1, x_v, out_v, psem, sem0, sem1):
    wid = lax.axis_index("s") * NC + lax.axis_index("c")
    base = wid * RPW

    idxs = [idx0, idx1]
    gaths = [gath0, gath1]
    sems = [sem0, sem1]
    copies = [None, None]

    def load_idx(c, buf):
        col0 = base + c * RC
        handles = [
            pltpu.async_copy(
                pt_hbm.at[l, pl.ds(col0, RC)], buf.at[pl.ds(l * RC, RC)], psem)
            for l in range(L)
        ]
        for h in handles:
            h.wait()

    pltpu.sync_copy(x_hbm.at[pl.ds(base, RPW)], x_v)
    load_idx(0, idx0)
    copies[0] = pltpu.async_copy(w_hbm.at[idx0], gath0, sem0)

    for c in range(NCHUNK):
        cur = c % 2
        nxt = (c + 1) % 2
        if c + 1 < NCHUNK:
            load_idx(c + 1, idxs[nxt])
            copies[nxt] = pltpu.async_copy(
                w_hbm.at[idxs[nxt]], gaths[nxt], sems[nxt])
        copies[cur].wait()
        gbuf = gaths[cur]

        def g_body(g, carry):
            row0 = pl.multiple_of(c * RC + g * 16, 16)
            g16 = pl.multiple_of(g * 16, 16)
            acc = x_v[pl.ds(row0, 16)]
            for l in range(L):
                acc = acc * gbuf[pl.ds(g16 + l * RC, 16)]
            out_v[pl.ds(row0, 16)] = acc
            return carry

        lax.fori_loop(0, RC // 16, g_body, 0)

    pltpu.sync_copy(out_v, out_hbm.at[pl.ds(base, RPW)])


def kernel(x, weights, paths):
    paths_t = paths.astype(jnp.int32).T
    mesh = plsc.VectorSubcoreMesh(core_axis_name="c", subcore_axis_name="s")
    f = pl.kernel(
        _body,
        out_type=jax.ShapeDtypeStruct((B,), jnp.float32),
        mesh=mesh,
        scratch_types=[
            pltpu.VMEM((IC,), jnp.int32),
            pltpu.VMEM((IC,), jnp.int32),
            pltpu.VMEM((IC,), jnp.float32),
            pltpu.VMEM((IC,), jnp.float32),
            pltpu.VMEM((RPW,), jnp.float32),
            pltpu.VMEM((RPW,), jnp.float32),
            pltpu.SemaphoreType.DMA,
            pltpu.SemaphoreType.DMA,
            pltpu.SemaphoreType.DMA,
        ],
        compiler_params=pltpu.CompilerParams(needs_layout_passes=False),
    )
    return f(x, weights, paths_t)


# speedup vs baseline: 1.2024x; 1.0237x over previous
import jax
import jax.numpy as jnp
from jax import lax
from jax.experimental import pallas as pl
from jax.experimental.pallas import tpu as pltpu
from jax.experimental.pallas import tpu_sc as plsc

B = 16384
L = 32
NC = 2
NS = 16
NW = NC * NS
RPW = B // NW
NCHUNK = 4
RC = RPW // NCHUNK
IC = RC * L


def _body(x_hbm, w_hbm, pt_hbm, out_hbm,
          idx0, idx1, gath0, gath1, x_v, out_v, psem, sem0, sem1):
    wid = lax.axis_index("s") * NC + lax.axis_index("c")
    base = wid * RPW

    idxs = [idx0, idx1]
    gaths = [gath0, gath1]
    sems = [sem0, sem1]
    copies = [None, None]

    def load_idx(c, buf):
        col0 = base + c * RC

        def fire(l, carry):
            pltpu.async_copy(
                pt_hbm.at[l, pl.ds(col0, RC)],
                buf.at[pl.ds(pl.multiple_of(l * RC, 8), RC)], psem)
            return carry

        def drain(l, carry):
            pltpu.make_async_copy(
                pt_hbm.at[l, pl.ds(col0, RC)],
                buf.at[pl.ds(pl.multiple_of(l * RC, 8), RC)], psem).wait()
            return carry

        lax.fori_loop(0, L, fire, 0)
        lax.fori_loop(0, L, drain, 0)

    pltpu.sync_copy(x_hbm.at[pl.ds(base, RPW)], x_v)
    load_idx(0, idx0)
    copies[0] = pltpu.async_copy(w_hbm.at[idx0], gath0, sem0)

    for c in range(NCHUNK):
        cur = c % 2
        nxt = (c + 1) % 2
        if c + 1 < NCHUNK:
            load_idx(c + 1, idxs[nxt])
            copies[nxt] = pltpu.async_copy(
                w_hbm.at[idxs[nxt]], gaths[nxt], sems[nxt])
        copies[cur].wait()
        gbuf = gaths[cur]

        def g_body(g, carry):
            row0 = pl.multiple_of(c * RC + g * 16, 16)
            g16 = pl.multiple_of(g * 16, 16)

            def l_body(l, acc):
                base4 = pl.multiple_of(l * (4 * RC), 8)
                for j in range(4):
                    acc = acc * gbuf[pl.ds(g16 + base4 + j * RC, 16)]
                return acc

            acc = lax.fori_loop(0, L // 4, l_body, x_v[pl.ds(row0, 16)])
            out_v[pl.ds(row0, 16)] = acc
            return carry

        lax.fori_loop(0, RC // 16, g_body, 0)

    pltpu.sync_copy(out_v, out_hbm.at[pl.ds(base, RPW)])


def kernel(x, weights, paths):
    paths_t = paths.astype(jnp.int32).T
    mesh = plsc.VectorSubcoreMesh(core_axis_name="c", subcore_axis_name="s")
    f = pl.kernel(
        _body,
        out_type=jax.ShapeDtypeStruct((B,), jnp.float32),
        mesh=mesh,
        scratch_types=[
            pltpu.VMEM((IC,), jnp.int32),
            pltpu.VMEM((IC,), jnp.int32),
            pltpu.VMEM((IC,), jnp.float32),
            pltpu.VMEM((IC,), jnp.float32),
            pltpu.VMEM((RPW,), jnp.float32),
            pltpu.VMEM((RPW,), jnp.float32),
            pltpu.SemaphoreType.DMA,
            pltpu.SemaphoreType.DMA,
            pltpu.SemaphoreType.DMA,
        ],
        compiler_params=pltpu.CompilerParams(needs_layout_passes=False),
    )
    return f(x, weights, paths_t)


# speedup vs baseline: 1.2086x; 1.0052x over previous
import jax
import jax.numpy as jnp
from jax import lax
from jax.experimental import pallas as pl
from jax.experimental.pallas import tpu as pltpu
from jax.experimental.pallas import tpu_sc as plsc

B = 16384
L = 32
NC = 2
NS = 16
NW = NC * NS
RPW = B // NW
NCHUNK = 4
RC = RPW // NCHUNK
IC = RC * L


def _body(x_hbm, w_hbm, pt_hbm, out_hbm,
          idx0, idx1, gath0, gath1, x_v, out_v,
          psem, sem0a, sem0b, sem1a, sem1b):
    wid = lax.axis_index("s") * NC + lax.axis_index("c")
    base = wid * RPW

    idxs = [idx0, idx1]
    gaths = [gath0, gath1]
    sems = [[sem0a, sem0b], [sem1a, sem1b]]
    copies = [None, None]

    def load_idx(c, buf):
        col0 = base + c * RC

        def fire(l, carry):
            pltpu.async_copy(
                pt_hbm.at[l, pl.ds(col0, RC)],
                buf.at[pl.ds(pl.multiple_of(l * RC, 8), RC)], psem)
            return carry

        def drain(l, carry):
            pltpu.make_async_copy(
                pt_hbm.at[l, pl.ds(col0, RC)],
                buf.at[pl.ds(pl.multiple_of(l * RC, 8), RC)], psem).wait()
            return carry

        lax.fori_loop(0, L, fire, 0)
        lax.fori_loop(0, L, drain, 0)

    HC = IC // 2

    def fire_gather(idx, gath, sem):
        return [
            pltpu.async_copy(
                w_hbm.at[idx.at[pl.ds(h * HC, HC)]],
                gath.at[pl.ds(h * HC, HC)], sem[h])
            for h in range(2)
        ]

    pltpu.sync_copy(x_hbm.at[pl.ds(base, RPW)], x_v)
    load_idx(0, idx0)
    copies[0] = fire_gather(idx0, gath0, sems[0])

    for c in range(NCHUNK):
        cur = c % 2
        nxt = (c + 1) % 2
        if c + 1 < NCHUNK:
            load_idx(c + 1, idxs[nxt])
            copies[nxt] = fire_gather(idxs[nxt], gaths[nxt], sems[nxt])
        for h in copies[cur]:
            h.wait()
        gbuf = gaths[cur]

        def g_body(g, carry):
            row0 = pl.multiple_of(c * RC + g * 16, 16)
            g16 = pl.multiple_of(g * 16, 16)

            def l_body(l, acc):
                base4 = pl.multiple_of(l * (4 * RC), 8)
                for j in range(4):
                    acc = acc * gbuf[pl.ds(g16 + base4 + j * RC, 16)]
                return acc

            acc = lax.fori_loop(0, L // 4, l_body, x_v[pl.ds(row0, 16)])
            out_v[pl.ds(row0, 16)] = acc
            return carry

        lax.fori_loop(0, RC // 16, g_body, 0)

    pltpu.sync_copy(out_v, out_hbm.at[pl.ds(base, RPW)])


def kernel(x, weights, paths):
    paths_t = paths.astype(jnp.int32).T
    mesh = plsc.VectorSubcoreMesh(core_axis_name="c", subcore_axis_name="s")
    f = pl.kernel(
        _body,
        out_type=jax.ShapeDtypeStruct((B,), jnp.float32),
        mesh=mesh,
        scratch_types=[
            pltpu.VMEM((IC,), jnp.int32),
            pltpu.VMEM((IC,), jnp.int32),
            pltpu.VMEM((IC,), jnp.float32),
            pltpu.VMEM((IC,), jnp.float32),
            pltpu.VMEM((RPW,), jnp.float32),
            pltpu.VMEM((RPW,), jnp.float32),
            pltpu.SemaphoreType.DMA,
            pltpu.SemaphoreType.DMA,
            pltpu.SemaphoreType.DMA,
            pltpu.SemaphoreType.DMA,
            pltpu.SemaphoreType.DMA,
        ],
        compiler_params=pltpu.CompilerParams(needs_layout_passes=False),
    )
    return f(x, weights, paths_t)


# speedup vs baseline: 1.2268x; 1.0151x over previous
import jax
import jax.numpy as jnp
from jax import lax
from jax.experimental import pallas as pl
from jax.experimental.pallas import tpu as pltpu
from jax.experimental.pallas import tpu_sc as plsc

B = 16384
L = 32
NC = 2
NS = 16
NW = NC * NS
RPW = B // NW
NCHUNK = 2
RC = RPW // NCHUNK
IC = RC * L


def _body(x_hbm, w_hbm, pt_hbm, out_hbm,
          idx0, idx1, gath0, gath1, x_v, out_v,
          psem, sem0a, sem0b, sem1a, sem1b):
    wid = lax.axis_index("s") * NC + lax.axis_index("c")
    base = wid * RPW

    idxs = [idx0, idx1]
    gaths = [gath0, gath1]
    sems = [[sem0a, sem0b], [sem1a, sem1b]]
    copies = [None, None]

    def load_idx(c, buf):
        col0 = base + c * RC

        def fire(l, carry):
            pltpu.async_copy(
                pt_hbm.at[l, pl.ds(col0, RC)],
                buf.at[pl.ds(pl.multiple_of(l * RC, 8), RC)], psem)
            return carry

        def drain(l, carry):
            pltpu.make_async_copy(
                pt_hbm.at[l, pl.ds(col0, RC)],
                buf.at[pl.ds(pl.multiple_of(l * RC, 8), RC)], psem).wait()
            return carry

        lax.fori_loop(0, L, fire, 0)
        lax.fori_loop(0, L, drain, 0)

    HC = IC // 2

    def fire_gather(idx, gath, sem):
        return [
            pltpu.async_copy(
                w_hbm.at[idx.at[pl.ds(h * HC, HC)]],
                gath.at[pl.ds(h * HC, HC)], sem[h])
            for h in range(2)
        ]

    pltpu.sync_copy(x_hbm.at[pl.ds(base, RPW)], x_v)
    load_idx(0, idx0)
    copies[0] = fire_gather(idx0, gath0, sems[0])

    for c in range(NCHUNK):
        cur = c % 2
        nxt = (c + 1) % 2
        if c + 1 < NCHUNK:
            load_idx(c + 1, idxs[nxt])
            copies[nxt] = fire_gather(idxs[nxt], gaths[nxt], sems[nxt])
        for h in copies[cur]:
            h.wait()
        gbuf = gaths[cur]

        def g_body(g, carry):
            row0 = pl.multiple_of(c * RC + g * 16, 16)
            g16 = pl.multiple_of(g * 16, 16)

            def l_body(l, acc):
                base4 = pl.multiple_of(l * (4 * RC), 8)
                for j in range(4):
                    acc = acc * gbuf[pl.ds(g16 + base4 + j * RC, 16)]
                return acc

            acc = lax.fori_loop(0, L // 4, l_body, x_v[pl.ds(row0, 16)])
            out_v[pl.ds(row0, 16)] = acc
            return carry

        lax.fori_loop(0, RC // 16, g_body, 0)

    pltpu.sync_copy(out_v, out_hbm.at[pl.ds(base, RPW)])


def kernel(x, weights, paths):
    paths_t = paths.astype(jnp.int32).T
    mesh = plsc.VectorSubcoreMesh(core_axis_name="c", subcore_axis_name="s")
    f = pl.kernel(
        _body,
        out_type=jax.ShapeDtypeStruct((B,), jnp.float32),
        mesh=mesh,
        scratch_types=[
            pltpu.VMEM((IC,), jnp.int32),
            pltpu.VMEM((IC,), jnp.int32),
            pltpu.VMEM((IC,), jnp.float32),
            pltpu.VMEM((IC,), jnp.float32),
            pltpu.VMEM((RPW,), jnp.float32),
            pltpu.VMEM((RPW,), jnp.float32),
            pltpu.SemaphoreType.DMA,
            pltpu.SemaphoreType.DMA,
            pltpu.SemaphoreType.DMA,
            pltpu.SemaphoreType.DMA,
            pltpu.SemaphoreType.DMA,
        ],
        compiler_params=pltpu.CompilerParams(needs_layout_passes=False),
    )
    return f(x, weights, paths_t)


# speedup vs baseline: 1.2418x; 1.0122x over previous
import jax
import jax.numpy as jnp
from jax import lax
from jax.experimental import pallas as pl
from jax.experimental.pallas import tpu as pltpu
from jax.experimental.pallas import tpu_sc as plsc

B = 16384
L = 32
NC = 2
NS = 16
NW = NC * NS
RPW = B // NW
NCHUNK = 2
RC = RPW // NCHUNK
IC = RC * L


def _body(x_hbm, w_hbm, pt_hbm, out_hbm,
          idx0, idx1, gath0, gath1, x_v, out_v,
          psem, sem0a, sem0b, sem1a, sem1b):
    wid = lax.axis_index("s") * NC + lax.axis_index("c")
    base = wid * RPW

    idxs = [idx0, idx1]
    gaths = [gath0, gath1]
    sems = [[sem0a, sem0b], [sem1a, sem1b]]
    copies = [None, None]

    def load_idx(c, buf):
        col0 = base + c * RC

        def fire(l, carry):
            pltpu.async_copy(
                pt_hbm.at[l, pl.ds(col0, RC)],
                buf.at[pl.ds(pl.multiple_of(l * RC, 8), RC)], psem)
            return carry

        lax.fori_loop(0, L, fire, 0)
        pltpu.make_async_copy(pt_hbm.at[0, pl.ds(0, IC)], buf, psem).wait()

    HC = IC // 2

    def fire_gather(idx, gath, sem):
        return [
            pltpu.async_copy(
                w_hbm.at[idx.at[pl.ds(h * HC, HC)]],
                gath.at[pl.ds(h * HC, HC)], sem[h])
            for h in range(2)
        ]

    pltpu.sync_copy(x_hbm.at[pl.ds(base, RPW)], x_v)
    load_idx(0, idx0)
    copies[0] = fire_gather(idx0, gath0, sems[0])

    for c in range(NCHUNK):
        cur = c % 2
        nxt = (c + 1) % 2
        if c + 1 < NCHUNK:
            load_idx(c + 1, idxs[nxt])
            copies[nxt] = fire_gather(idxs[nxt], gaths[nxt], sems[nxt])
        for h in copies[cur]:
            h.wait()
        gbuf = gaths[cur]

        def g_body(g, carry):
            row0 = pl.multiple_of(c * RC + g * 16, 16)
            g16 = pl.multiple_of(g * 16, 16)

            def l_body(l, acc):
                base4 = pl.multiple_of(l * (4 * RC), 8)
                for j in range(4):
                    acc = acc * gbuf[pl.ds(g16 + base4 + j * RC, 16)]
                return acc

            acc = lax.fori_loop(0, L // 4, l_body, x_v[pl.ds(row0, 16)])
            out_v[pl.ds(row0, 16)] = acc
            return carry

        lax.fori_loop(0, RC // 16, g_body, 0)

    pltpu.sync_copy(out_v, out_hbm.at[pl.ds(base, RPW)])


def kernel(x, weights, paths):
    paths_t = paths.astype(jnp.int32).T
    mesh = plsc.VectorSubcoreMesh(core_axis_name="c", subcore_axis_name="s")
    f = pl.kernel(
        _body,
        out_type=jax.ShapeDtypeStruct((B,), jnp.float32),
        mesh=mesh,
        scratch_types=[
            pltpu.VMEM((IC,), jnp.int32),
            pltpu.VMEM((IC,), jnp.int32),
            pltpu.VMEM((IC,), jnp.float32),
            pltpu.VMEM((IC,), jnp.float32),
            pltpu.VMEM((RPW,), jnp.float32),
            pltpu.VMEM((RPW,), jnp.float32),
            pltpu.SemaphoreType.DMA,
            pltpu.SemaphoreType.DMA,
            pltpu.SemaphoreType.DMA,
            pltpu.SemaphoreType.DMA,
            pltpu.SemaphoreType.DMA,
        ],
        compiler_params=pltpu.CompilerParams(needs_layout_passes=False),
    )
    return f(x, weights, paths_t)


# speedup vs baseline: 1.2482x; 1.0051x over previous
import jax
import jax.numpy as jnp
from jax import lax
from jax.experimental import pallas as pl
from jax.experimental.pallas import tpu as pltpu
from jax.experimental.pallas import tpu_sc as plsc

B = 16384
L = 32
NC = 2
NS = 16
NW = NC * NS
RPW = B // NW
NCHUNK = 2
RC = RPW // NCHUNK
IC = RC * L


def _body(x_hbm, w_hbm, pt_hbm, out_hbm,
          idx0, idx1, gath0, gath1, x_v, out_v,
          psem, sem0a, sem0b, sem1a, sem1b):
    wid = lax.axis_index("s") * NC + lax.axis_index("c")
    base = wid * RPW

    idxs = [idx0, idx1]
    gaths = [gath0, gath1]
    sems = [[sem0a, sem0b], [sem1a, sem1b]]
    copies = [None, None]

    def load_idx(c, buf):
        col0 = base + c * RC

        def fire(l, carry):
            pltpu.async_copy(
                pt_hbm.at[l, pl.ds(col0, RC)],
                buf.at[pl.ds(pl.multiple_of(l * RC, 8), RC)], psem)
            return carry

        lax.fori_loop(0, L, fire, 0)
        pltpu.make_async_copy(pt_hbm.at[0, pl.ds(0, IC)], buf, psem).wait()

    HC = IC // 2

    def fire_gather(idx, gath, sem):
        return [
            pltpu.async_copy(
                w_hbm.at[idx.at[pl.ds(h * HC, HC)]],
                gath.at[pl.ds(h * HC, HC)], sem[h])
            for h in range(2)
        ]

    pltpu.sync_copy(x_hbm.at[pl.ds(base, RPW)], x_v)
    load_idx(0, idx0)
    copies[0] = fire_gather(idx0, gath0, sems[0])

    for c in range(NCHUNK):
        cur = c % 2
        nxt = (c + 1) % 2
        if c + 1 < NCHUNK:
            load_idx(c + 1, idxs[nxt])
            copies[nxt] = fire_gather(idxs[nxt], gaths[nxt], sems[nxt])
        gbuf = gaths[cur]

        for h in range(2):
            copies[cur][h].wait()

            def g_body(g, carry):
                row0 = pl.multiple_of(c * RC + g * 16, 16)
                g16 = pl.multiple_of(g * 16, 16)

                def l_body(l, acc):
                    base4 = pl.multiple_of(h * HC + l * (4 * RC), 8)
                    for j in range(4):
                        acc = acc * gbuf[pl.ds(g16 + base4 + j * RC, 16)]
                    return acc

                init = x_v[pl.ds(row0, 16)] if h == 0 else out_v[pl.ds(row0, 16)]
                acc = lax.fori_loop(0, L // 8, l_body, init)
                out_v[pl.ds(row0, 16)] = acc
                return carry

            lax.fori_loop(0, RC // 16, g_body, 0)

    pltpu.sync_copy(out_v, out_hbm.at[pl.ds(base, RPW)])


def kernel(x, weights, paths):
    paths_t = paths.astype(jnp.int32).T
    mesh = plsc.VectorSubcoreMesh(core_axis_name="c", subcore_axis_name="s")
    f = pl.kernel(
        _body,
        out_type=jax.ShapeDtypeStruct((B,), jnp.float32),
        mesh=mesh,
        scratch_types=[
            pltpu.VMEM((IC,), jnp.int32),
            pltpu.VMEM((IC,), jnp.int32),
            pltpu.VMEM((IC,), jnp.float32),
            pltpu.VMEM((IC,), jnp.float32),
            pltpu.VMEM((RPW,), jnp.float32),
            pltpu.VMEM((RPW,), jnp.float32),
            pltpu.SemaphoreType.DMA,
            pltpu.SemaphoreType.DMA,
            pltpu.SemaphoreType.DMA,
            pltpu.SemaphoreType.DMA,
            pltpu.SemaphoreType.DMA,
        ],
        compiler_params=pltpu.CompilerParams(needs_layout_passes=False),
    )
    return f(x, weights, paths_t)


# speedup vs baseline: 1.2524x; 1.0034x over previous
import jax
import jax.numpy as jnp
from jax import lax
from jax.experimental import pallas as pl
from jax.experimental.pallas import tpu as pltpu
from jax.experimental.pallas import tpu_sc as plsc

B = 16384
L = 32
NC = 2
NS = 16
NW = NC * NS
RPW = B // NW
NCHUNK = 2
RC = RPW // NCHUNK
IC = RC * L


def _body(x_hbm, w_hbm, pt_hbm, out_hbm,
          idx0, idx1, gath0, gath1, x_v, out_v,
          psem, sem0a, sem0b, sem1a, sem1b):
    wid = lax.axis_index("s") * NC + lax.axis_index("c")
    base = wid * RPW

    idxs = [idx0, idx1]
    gaths = [gath0, gath1]
    sems = [[sem0a, sem0b], [sem1a, sem1b]]
    copies = [None, None]

    HC = IC // 2

    def load_idx_half(c, buf, h):
        col0 = base + c * RC

        def fire(l, carry):
            pltpu.async_copy(
                pt_hbm.at[l, pl.ds(col0, RC)],
                buf.at[pl.ds(pl.multiple_of(l * RC, 8), RC)], psem)
            return carry

        lax.fori_loop(h * (L // 2), (h + 1) * (L // 2), fire, 0)
        pltpu.make_async_copy(
            pt_hbm.at[0, pl.ds(0, HC)], buf.at[pl.ds(h * HC, HC)], psem).wait()

    def stage_chunk(c, buf, gath, sem):
        out = []
        for h in range(2):
            load_idx_half(c, buf, h)
            out.append(pltpu.async_copy(
                w_hbm.at[buf.at[pl.ds(h * HC, HC)]],
                gath.at[pl.ds(h * HC, HC)], sem[h]))
        return out

    pltpu.sync_copy(x_hbm.at[pl.ds(base, RPW)], x_v)
    copies[0] = stage_chunk(0, idx0, gath0, sems[0])

    for c in range(NCHUNK):
        cur = c % 2
        nxt = (c + 1) % 2
        if c + 1 < NCHUNK:
            copies[nxt] = stage_chunk(c + 1, idxs[nxt], gaths[nxt], sems[nxt])
        gbuf = gaths[cur]

        for h in range(2):
            copies[cur][h].wait()

            def g_body(g, carry):
                row0 = pl.multiple_of(c * RC + g * 16, 16)
                g16 = pl.multiple_of(g * 16, 16)

                def l_body(l, acc):
                    base4 = pl.multiple_of(h * HC + l * (4 * RC), 8)
                    for j in range(4):
                        acc = acc * gbuf[pl.ds(g16 + base4 + j * RC, 16)]
                    return acc

                init = x_v[pl.ds(row0, 16)] if h == 0 else out_v[pl.ds(row0, 16)]
                acc = lax.fori_loop(0, L // 8, l_body, init)
                out_v[pl.ds(row0, 16)] = acc
                return carry

            lax.fori_loop(0, RC // 16, g_body, 0)

    pltpu.sync_copy(out_v, out_hbm.at[pl.ds(base, RPW)])


def kernel(x, weights, paths):
    paths_t = paths.astype(jnp.int32).T
    mesh = plsc.VectorSubcoreMesh(core_axis_name="c", subcore_axis_name="s")
    f = pl.kernel(
        _body,
        out_type=jax.ShapeDtypeStruct((B,), jnp.float32),
        mesh=mesh,
        scratch_types=[
            pltpu.VMEM((IC,), jnp.int32),
            pltpu.VMEM((IC,), jnp.int32),
            pltpu.VMEM((IC,), jnp.float32),
            pltpu.VMEM((IC,), jnp.float32),
            pltpu.VMEM((RPW,), jnp.float32),
            pltpu.VMEM((RPW,), jnp.float32),
            pltpu.SemaphoreType.DMA,
            pltpu.SemaphoreType.DMA,
            pltpu.SemaphoreType.DMA,
            pltpu.SemaphoreType.DMA,
            pltpu.SemaphoreType.DMA,
        ],
        compiler_params=pltpu.CompilerParams(needs_layout_passes=False),
    )
    return f(x, weights, paths_t)
